# Initial kernel scaffold; baseline (speedup 1.0000x reference)
import jax
import jax.numpy as jnp
from jax.experimental import pallas as pl


def kernel(x, w_p1, g_p1, b_p1, m_p1, v_p1, w_p2, g_p2, b_p2, m_p2, v_p2, w_p, g_pbn, b_pbn, m_pbn, v_pbn, w_c1, g_bn1, b_bn1, m_bn1, v_bn1, w_c2, g_c2, b_c2, m_c2, v_c2):
    raise NotImplementedError("write your pallas kernel here")


# speedup vs baseline: 2.1463x; 2.1463x over previous
import jax
import jax.numpy as jnp
from jax.experimental import pallas as pl
from jax.experimental.pallas import tpu as pltpu

EPS = 1e-3
B, C, H, W = 8, 256, 128, 128
TH = 32
NT = H // TH


def _k1(xp_ref, wb_ref, bb_ref, cm_ref, rm_ref):
    i = pl.program_id(1)
    h0 = i * TH
    xs = xp_ref[0, pl.ds(h0, TH + 2), :, :]
    acc = jnp.zeros((TH * W, 256), jnp.float32)
    for ky in range(3):
        for kx in range(3):
            blk = xs[ky:ky + TH, kx:kx + W, :].reshape(TH * W, C)
            acc += jnp.dot(blk, wb_ref[3 * ky + kx],
                           preferred_element_type=jnp.float32)
    p = jnp.maximum(acc + bb_ref[0], 0.0).reshape(TH, W, 256)
    cm = jnp.max(p[:, :, :128], axis=0)
    rm = jnp.max(p[:, :, 128:], axis=1)
    rm_ref[0] = rm

    @pl.when(i == 0)
    def _():
        cm_ref[0] = cm

    @pl.when(i > 0)
    def _():
        cm_ref[0] = jnp.maximum(cm_ref[0], cm)


def _k2(xp_ref, cm_ref, rmp_ref, wm_ref, bm_ref, ws_ref, bs_ref,
        wo_ref, bo_ref, o_ref):
    i = pl.program_id(1)
    h0 = i * TH

    rowvals = rmp_ref[0, pl.ds(h0, TH + 4), :]
    r_img = h0 - 2 + jax.lax.broadcasted_iota(jnp.int32, (TH + 4, 1, 1), 0)
    rmask = ((r_img >= 0) & (r_img < H)).astype(jnp.float32)
    colpad = jnp.pad(cm_ref[0], ((1, 1), (0, 0)))
    c_pad = jax.lax.broadcasted_iota(jnp.int32, (1, W + 2, 1), 1)
    cmask = ((c_pad >= 1) & (c_pad <= W)).astype(jnp.float32)
    mtile = (rowvals[:, None, :] + colpad[None, :, :]) * (rmask * cmask)

    acc = jnp.zeros(((TH + 2) * W, 256), jnp.float32)
    for ky in range(3):
        for kx in range(3):
            blk = mtile[ky:ky + TH + 2, kx:kx + W, :].reshape((TH + 2) * W, 128)
            acc += jnp.dot(blk, wm_ref[3 * ky + kx],
                           preferred_element_type=jnp.float32)
    pbn1 = acc + bm_ref[0]

    xsk = xp_ref[0, pl.ds(h0, TH + 2), pl.ds(1, W), :].reshape((TH + 2) * W, C)
    bn1 = jnp.dot(xsk, ws_ref[...], preferred_element_type=jnp.float32) + bs_ref[0]

    relu1 = jnp.maximum(pbn1 + bn1, 0.0).reshape(TH + 2, W, 256)
    y_img = h0 - 1 + jax.lax.broadcasted_iota(jnp.int32, (TH + 2, 1, 1), 0)
    ymask = ((y_img >= 0) & (y_img < H)).astype(jnp.float32)
    r1p = jnp.pad(relu1 * ymask, ((0, 0), (1, 1), (0, 0)))

    acc2 = jnp.zeros((TH * W, 256), jnp.float32)
    for ky in range(3):
        for kx in range(3):
            blk = r1p[ky:ky + TH, kx:kx + W, :].reshape(TH * W, 256)
            acc2 += jnp.dot(blk, wo_ref[3 * ky + kx],
                            preferred_element_type=jnp.float32)
    o_ref[0] = jnp.maximum(acc2 + bo_ref[0], 0.0).reshape(TH, W, 256)


def _fold_bn(w, g, b, m, v):
    s = g / jnp.sqrt(v + EPS)
    w2 = (w * s[:, None, None, None]).transpose(2, 3, 1, 0)
    return w2, (b - m * s)[None, :]


def kernel(x, w_p1, g_p1, b_p1, m_p1, v_p1, w_p2, g_p2, b_p2, m_p2, v_p2,
           w_p, g_pbn, b_pbn, m_pbn, v_pbn, w_c1, g_bn1, b_bn1, m_bn1, v_bn1,
           w_c2, g_c2, b_c2, m_c2, v_c2):
    xp = jnp.pad(x.transpose(0, 2, 3, 1), ((0, 0), (1, 1), (1, 1), (0, 0)))

    w1, bb1 = _fold_bn(w_p1, g_p1, b_p1, m_p1, v_p1)
    w2, bb2 = _fold_bn(w_p2, g_p2, b_p2, m_p2, v_p2)
    wb = jnp.concatenate([w1, w2], axis=-1).reshape(9, C, 256)
    bb = jnp.concatenate([bb1, bb2], axis=-1)
    wm, bm = _fold_bn(w_p, g_pbn, b_pbn, m_pbn, v_pbn)
    wm = wm.reshape(9, 128, 256)
    ws, bs = _fold_bn(w_c1, g_bn1, b_bn1, m_bn1, v_bn1)
    ws = ws.reshape(C, 256)
    wo, bo = _fold_bn(w_c2, g_c2, b_c2, m_c2, v_c2)
    wo = wo.reshape(9, C, 256)

    cparams = pltpu.CompilerParams(
        dimension_semantics=("parallel", "arbitrary"),
        vmem_limit_bytes=64 * 1024 * 1024,
    )

    full = lambda *shape: pl.BlockSpec(shape, lambda b_, i: (0,) * len(shape))

    cm, rm = pl.pallas_call(
        _k1,
        grid=(B, NT),
        in_specs=[
            pl.BlockSpec((1, H + 2, W + 2, C), lambda b_, i: (b_, 0, 0, 0)),
            full(9, C, 256),
            full(1, 256),
        ],
        out_specs=[
            pl.BlockSpec((1, W, 128), lambda b_, i: (b_, 0, 0)),
            pl.BlockSpec((1, TH, 128), lambda b_, i: (b_, i, 0)),
        ],
        out_shape=[
            jax.ShapeDtypeStruct((B, W, 128), jnp.float32),
            jax.ShapeDtypeStruct((B, H, 128), jnp.float32),
        ],
        compiler_params=cparams,
    )(xp, wb, bb)

    rmp = jnp.pad(rm, ((0, 0), (2, 2), (0, 0)))

    out_t = pl.pallas_call(
        _k2,
        grid=(B, NT),
        in_specs=[
            pl.BlockSpec((1, H + 2, W + 2, C), lambda b_, i: (b_, 0, 0, 0)),
            pl.BlockSpec((1, W, 128), lambda b_, i: (b_, 0, 0)),
            pl.BlockSpec((1, H + 4, 128), lambda b_, i: (b_, 0, 0)),
            full(9, 128, 256),
            full(1, 256),
            full(C, 256),
            full(1, 256),
            full(9, C, 256),
            full(1, 256),
        ],
        out_specs=pl.BlockSpec((1, TH, W, 256), lambda b_, i: (b_, i, 0, 0)),
        out_shape=jax.ShapeDtypeStruct((B, H, W, 256), jnp.float32),
        compiler_params=cparams,
    )(xp, cm, rmp, wm, bm, ws, bs, wo, bo)

    return out_t.transpose(0, 3, 1, 2)


# speedup vs baseline: 2.1702x; 1.0111x over previous
import jax
import jax.numpy as jnp
from jax.experimental import pallas as pl
from jax.experimental.pallas import tpu as pltpu

EPS = 1e-3
B, C, H, W = 8, 256, 128, 128
TH = 32
NT = H // TH


def _k1(xp_ref, wb_ref, bb_ref, cm_ref, rm_ref):
    i = pl.program_id(1)
    h0 = i * TH
    xs = xp_ref[0, pl.ds(h0, TH + 2), :, :]
    acc = jnp.zeros((TH * W, 256), jnp.float32)
    for ky in range(3):
        for kx in range(3):
            blk = xs[ky:ky + TH, kx:kx + W, :].reshape(TH * W, C)
            acc += jnp.dot(blk, wb_ref[3 * ky + kx],
                           preferred_element_type=jnp.float32)
    p = jnp.maximum(acc + bb_ref[0], 0.0).reshape(TH, W, 256)
    cm = jnp.max(p[:, :, :128], axis=0)
    rm = jnp.max(p[:, :, 128:], axis=1)
    rm_ref[0] = rm

    @pl.when(i == 0)
    def _():
        cm_ref[0] = cm

    @pl.when(i > 0)
    def _():
        cm_ref[0] = jnp.maximum(cm_ref[0], cm)


def _k2(xp_ref, cm_ref, rmp_ref, wm_ref, bm_ref, ws_ref, bs_ref,
        wo_ref, bo_ref, o_ref):
    i = pl.program_id(1)
    h0 = i * TH

    rowvals = rmp_ref[0, pl.ds(h0, TH + 4), :]
    r_img = h0 - 2 + jax.lax.broadcasted_iota(jnp.int32, (TH + 4, 1, 1), 0)
    rmask = ((r_img >= 0) & (r_img < H)).astype(jnp.float32)
    colpad = jnp.pad(cm_ref[0], ((1, 1), (0, 0)))
    c_pad = jax.lax.broadcasted_iota(jnp.int32, (1, W + 2, 1), 1)
    cmask = ((c_pad >= 1) & (c_pad <= W)).astype(jnp.float32)
    mtile = ((rowvals[:, None, :] + colpad[None, :, :])
             * (rmask * cmask)).astype(jnp.bfloat16)

    acc = jnp.zeros(((TH + 2) * W, 256), jnp.float32)
    for ky in range(3):
        for kx in range(3):
            blk = mtile[ky:ky + TH + 2, kx:kx + W, :].reshape((TH + 2) * W, 128)
            acc += jnp.dot(blk, wm_ref[3 * ky + kx],
                           preferred_element_type=jnp.float32)
    pbn1 = acc + bm_ref[0]

    xsk = xp_ref[0, pl.ds(h0, TH + 2), pl.ds(1, W), :].reshape((TH + 2) * W, C)
    bn1 = jnp.dot(xsk, ws_ref[...], preferred_element_type=jnp.float32) + bs_ref[0]

    relu1 = jnp.maximum(pbn1 + bn1, 0.0).reshape(TH + 2, W, 256)
    y_img = h0 - 1 + jax.lax.broadcasted_iota(jnp.int32, (TH + 2, 1, 1), 0)
    ymask = ((y_img >= 0) & (y_img < H)).astype(jnp.float32)
    r1p = jnp.pad((relu1 * ymask).astype(jnp.bfloat16),
                  ((0, 0), (1, 1), (0, 0)))

    acc2 = jnp.zeros((TH * W, 256), jnp.float32)
    for ky in range(3):
        for kx in range(3):
            blk = r1p[ky:ky + TH, kx:kx + W, :].reshape(TH * W, 256)
            acc2 += jnp.dot(blk, wo_ref[3 * ky + kx],
                            preferred_element_type=jnp.float32)
    o_ref[0] = jnp.maximum(acc2 + bo_ref[0], 0.0).reshape(TH, W, 256)


def _fold_bn(w, g, b, m, v):
    s = g / jnp.sqrt(v + EPS)
    w2 = (w * s[:, None, None, None]).transpose(2, 3, 1, 0).astype(jnp.bfloat16)
    return w2, (b - m * s)[None, :]


def kernel(x, w_p1, g_p1, b_p1, m_p1, v_p1, w_p2, g_p2, b_p2, m_p2, v_p2,
           w_p, g_pbn, b_pbn, m_pbn, v_pbn, w_c1, g_bn1, b_bn1, m_bn1, v_bn1,
           w_c2, g_c2, b_c2, m_c2, v_c2):
    xp = jnp.pad(x.transpose(0, 2, 3, 1).astype(jnp.bfloat16),
                 ((0, 0), (1, 1), (1, 1), (0, 0)))

    w1, bb1 = _fold_bn(w_p1, g_p1, b_p1, m_p1, v_p1)
    w2, bb2 = _fold_bn(w_p2, g_p2, b_p2, m_p2, v_p2)
    wb = jnp.concatenate([w1, w2], axis=-1).reshape(9, C, 256)
    bb = jnp.concatenate([bb1, bb2], axis=-1)
    wm, bm = _fold_bn(w_p, g_pbn, b_pbn, m_pbn, v_pbn)
    wm = wm.reshape(9, 128, 256)
    ws, bs = _fold_bn(w_c1, g_bn1, b_bn1, m_bn1, v_bn1)
    ws = ws.reshape(C, 256)
    wo, bo = _fold_bn(w_c2, g_c2, b_c2, m_c2, v_c2)
    wo = wo.reshape(9, C, 256)

    cparams = pltpu.CompilerParams(
        dimension_semantics=("parallel", "arbitrary"),
        vmem_limit_bytes=64 * 1024 * 1024,
    )

    full = lambda *shape: pl.BlockSpec(shape, lambda b_, i: (0,) * len(shape))

    cm, rm = pl.pallas_call(
        _k1,
        grid=(B, NT),
        in_specs=[
            pl.BlockSpec((1, H + 2, W + 2, C), lambda b_, i: (b_, 0, 0, 0)),
            full(9, C, 256),
            full(1, 256),
        ],
        out_specs=[
            pl.BlockSpec((1, W, 128), lambda b_, i: (b_, 0, 0)),
            pl.BlockSpec((1, TH, 128), lambda b_, i: (b_, i, 0)),
        ],
        out_shape=[
            jax.ShapeDtypeStruct((B, W, 128), jnp.float32),
            jax.ShapeDtypeStruct((B, H, 128), jnp.float32),
        ],
        compiler_params=cparams,
    )(xp, wb, bb)

    rmp = jnp.pad(rm, ((0, 0), (2, 2), (0, 0)))

    out_t = pl.pallas_call(
        _k2,
        grid=(B, NT),
        in_specs=[
            pl.BlockSpec((1, H + 2, W + 2, C), lambda b_, i: (b_, 0, 0, 0)),
            pl.BlockSpec((1, W, 128), lambda b_, i: (b_, 0, 0)),
            pl.BlockSpec((1, H + 4, 128), lambda b_, i: (b_, 0, 0)),
            full(9, 128, 256),
            full(1, 256),
            full(C, 256),
            full(1, 256),
            full(9, C, 256),
            full(1, 256),
        ],
        out_specs=pl.BlockSpec((1, TH, W, 256), lambda b_, i: (b_, i, 0, 0)),
        out_shape=jax.ShapeDtypeStruct((B, H, W, 256), jnp.float32),
        compiler_params=cparams,
    )(xp, cm, rmp, wm, bm, ws, bs, wo, bo)

    return out_t.transpose(0, 3, 1, 2)


# speedup vs baseline: 2.5408x; 1.1708x over previous
import jax
import jax.numpy as jnp
from jax.experimental import pallas as pl
from jax.experimental.pallas import tpu as pltpu

EPS = 1e-3
B, C, H, W = 8, 256, 128, 128
TH = 32
NT = H // TH


def _k1(xp_ref, wb_ref, bb_ref, cm_ref, rm_ref):
    i = pl.program_id(1)
    h0 = i * TH
    xs = xp_ref[0, pl.ds(h0, TH + 2), :, :]
    cat = jnp.concatenate(
        [xs[ky:ky + TH, kx:kx + W, :] for ky in range(3) for kx in range(3)],
        axis=-1).reshape(TH * W, 9 * C)
    p = jnp.dot(cat, wb_ref[...], preferred_element_type=jnp.float32)
    p = jnp.maximum(p + bb_ref[0], 0.0).reshape(TH, W, 256)
    cm = jnp.max(p[:, :, :128], axis=0)
    rm = jnp.max(p[:, :, 128:], axis=1)
    rm_ref[0] = rm

    @pl.when(i == 0)
    def _():
        cm_ref[0] = cm

    @pl.when(i > 0)
    def _():
        cm_ref[0] = jnp.maximum(cm_ref[0], cm)


def _k2(xp_ref, cm_ref, rmp_ref, wm_ref, bm_ref, ws_ref, bs_ref,
        wo_ref, bo_ref, o_ref):
    i = pl.program_id(1)
    h0 = i * TH

    rowvals = rmp_ref[0, pl.ds(h0, TH + 4), :]
    r_img = h0 - 2 + jax.lax.broadcasted_iota(jnp.int32, (TH + 4, 1, 1), 0)
    rmask = ((r_img >= 0) & (r_img < H)).astype(jnp.float32)
    colpad = jnp.pad(cm_ref[0], ((1, 1), (0, 0)))
    c_pad = jax.lax.broadcasted_iota(jnp.int32, (1, W + 2, 1), 1)
    cmask = ((c_pad >= 1) & (c_pad <= W)).astype(jnp.float32)
    mtile = ((rowvals[:, None, :] + colpad[None, :, :])
             * (rmask * cmask)).astype(jnp.bfloat16)

    mcat = jnp.concatenate(
        [mtile[ky:ky + TH + 2, kx:kx + W, :] for ky in range(3) for kx in range(3)],
        axis=-1).reshape((TH + 2) * W, 9 * 128)
    pbn1 = jnp.dot(mcat, wm_ref[...], preferred_element_type=jnp.float32) + bm_ref[0]

    xsk = xp_ref[0, pl.ds(h0, TH + 2), pl.ds(1, W), :].reshape((TH + 2) * W, C)
    bn1 = jnp.dot(xsk, ws_ref[...], preferred_element_type=jnp.float32) + bs_ref[0]

    relu1 = jnp.maximum(pbn1 + bn1, 0.0).reshape(TH + 2, W, 256)
    y_img = h0 - 1 + jax.lax.broadcasted_iota(jnp.int32, (TH + 2, 1, 1), 0)
    ymask = ((y_img >= 0) & (y_img < H)).astype(jnp.float32)
    r1p = jnp.pad((relu1 * ymask).astype(jnp.bfloat16),
                  ((0, 0), (1, 1), (0, 0)))

    ocat = jnp.concatenate(
        [r1p[ky:ky + TH, kx:kx + W, :] for ky in range(3) for kx in range(3)],
        axis=-1).reshape(TH * W, 9 * C)
    acc2 = jnp.dot(ocat, wo_ref[...], preferred_element_type=jnp.float32)
    o_ref[0] = jnp.maximum(acc2 + bo_ref[0], 0.0).reshape(TH, W, 256)


def _fold_bn(w, g, b, m, v):
    s = g / jnp.sqrt(v + EPS)
    w2 = (w * s[:, None, None, None]).transpose(2, 3, 1, 0).astype(jnp.bfloat16)
    return w2, (b - m * s)[None, :]


def kernel(x, w_p1, g_p1, b_p1, m_p1, v_p1, w_p2, g_p2, b_p2, m_p2, v_p2,
           w_p, g_pbn, b_pbn, m_pbn, v_pbn, w_c1, g_bn1, b_bn1, m_bn1, v_bn1,
           w_c2, g_c2, b_c2, m_c2, v_c2):
    xp = jnp.pad(x.transpose(0, 2, 3, 1).astype(jnp.bfloat16),
                 ((0, 0), (1, 1), (1, 1), (0, 0)))

    w1, bb1 = _fold_bn(w_p1, g_p1, b_p1, m_p1, v_p1)
    w2, bb2 = _fold_bn(w_p2, g_p2, b_p2, m_p2, v_p2)
    wb = jnp.concatenate([w1, w2], axis=-1).reshape(9 * C, 256)
    bb = jnp.concatenate([bb1, bb2], axis=-1)
    wm, bm = _fold_bn(w_p, g_pbn, b_pbn, m_pbn, v_pbn)
    wm = wm.reshape(9 * 128, 256)
    ws, bs = _fold_bn(w_c1, g_bn1, b_bn1, m_bn1, v_bn1)
    ws = ws.reshape(C, 256)
    wo, bo = _fold_bn(w_c2, g_c2, b_c2, m_c2, v_c2)
    wo = wo.reshape(9 * C, 256)

    cparams = pltpu.CompilerParams(
        dimension_semantics=("parallel", "arbitrary"),
        vmem_limit_bytes=64 * 1024 * 1024,
    )

    full = lambda *shape: pl.BlockSpec(shape, lambda b_, i: (0,) * len(shape))

    cm, rm = pl.pallas_call(
        _k1,
        grid=(B, NT),
        in_specs=[
            pl.BlockSpec((1, H + 2, W + 2, C), lambda b_, i: (b_, 0, 0, 0)),
            full(9 * C, 256),
            full(1, 256),
        ],
        out_specs=[
            pl.BlockSpec((1, W, 128), lambda b_, i: (b_, 0, 0)),
            pl.BlockSpec((1, TH, 128), lambda b_, i: (b_, i, 0)),
        ],
        out_shape=[
            jax.ShapeDtypeStruct((B, W, 128), jnp.float32),
            jax.ShapeDtypeStruct((B, H, 128), jnp.float32),
        ],
        compiler_params=cparams,
    )(xp, wb, bb)

    rmp = jnp.pad(rm, ((0, 0), (2, 2), (0, 0)))

    out_t = pl.pallas_call(
        _k2,
        grid=(B, NT),
        in_specs=[
            pl.BlockSpec((1, H + 2, W + 2, C), lambda b_, i: (b_, 0, 0, 0)),
            pl.BlockSpec((1, W, 128), lambda b_, i: (b_, 0, 0)),
            pl.BlockSpec((1, H + 4, 128), lambda b_, i: (b_, 0, 0)),
            full(9 * 128, 256),
            full(1, 256),
            full(C, 256),
            full(1, 256),
            full(9 * C, 256),
            full(1, 256),
        ],
        out_specs=pl.BlockSpec((1, TH, W, 256), lambda b_, i: (b_, i, 0, 0)),
        out_shape=jax.ShapeDtypeStruct((B, H, W, 256), jnp.float32),
        compiler_params=cparams,
    )(xp, cm, rmp, wm, bm, ws, bs, wo, bo)

    return out_t.transpose(0, 3, 1, 2)


# speedup vs baseline: 2.8574x; 1.1246x over previous
import jax
import jax.numpy as jnp
from jax.experimental import pallas as pl
from jax.experimental.pallas import tpu as pltpu

EPS = 1e-3
B, C, H, W = 8, 256, 128, 128
TH = 32
NT = H // TH


def _k1(xp_ref, wb_ref, bb_ref, cm_ref, rm_ref):
    i = pl.program_id(1)
    h0 = i * TH
    xs = xp_ref[0, pl.ds(h0, TH + 2), :, :]
    cat = jnp.concatenate(
        [xs[ky:ky + TH, kx:kx + W, :] for ky in range(3) for kx in range(3)],
        axis=-1).reshape(TH * W, 9 * C)
    p = jnp.dot(cat, wb_ref[...], preferred_element_type=jnp.float32)
    p = jnp.maximum(p + bb_ref[0], 0.0).reshape(TH, W, 256)
    cm = jnp.max(p[:, :, :128], axis=0)
    rm = jnp.max(p[:, :, 128:], axis=1)
    rm_ref[0] = rm

    @pl.when(i == 0)
    def _():
        cm_ref[0] = cm

    @pl.when(i > 0)
    def _():
        cm_ref[0] = jnp.maximum(cm_ref[0], cm)


def _k2(xp_ref, cm_ref, rmp_ref, wm_ref, bm_ref, ws_ref, bs_ref,
        wo_ref, bo_ref, o_ref):
    i = pl.program_id(1)
    h0 = i * TH

    rowvals = rmp_ref[0, pl.ds(h0, TH + 4), :]
    r_img = h0 - 2 + jax.lax.broadcasted_iota(jnp.int32, (TH + 4, 1, 1), 0)
    rmask = ((r_img >= 0) & (r_img < H)).astype(jnp.float32)
    colpad = jnp.pad(cm_ref[0], ((1, 1), (0, 0)))
    c_pad = jax.lax.broadcasted_iota(jnp.int32, (1, W + 2, 1), 1)
    cmask = ((c_pad >= 1) & (c_pad <= W)).astype(jnp.float32)
    mtile = ((rowvals[:, None, :] + colpad[None, :, :])
             * (rmask * cmask)).astype(jnp.bfloat16)

    mcat = jnp.concatenate(
        [mtile[ky:ky + TH + 2, kx:kx + W, :] for ky in range(3) for kx in range(3)],
        axis=-1).reshape((TH + 2) * W, 9 * 128)
    pbn1 = jnp.dot(mcat, wm_ref[...], preferred_element_type=jnp.float32) + bm_ref[0]

    xsk = xp_ref[0, pl.ds(h0, TH + 2), pl.ds(1, W), :].reshape((TH + 2) * W, C)
    bn1 = jnp.dot(xsk, ws_ref[...], preferred_element_type=jnp.float32) + bs_ref[0]

    relu1 = jnp.maximum(pbn1 + bn1, 0.0).reshape(TH + 2, W, 256)
    y_img = h0 - 1 + jax.lax.broadcasted_iota(jnp.int32, (TH + 2, 1, 1), 0)
    ymask = ((y_img >= 0) & (y_img < H)).astype(jnp.float32)
    r1p = jnp.pad((relu1 * ymask).astype(jnp.bfloat16),
                  ((0, 0), (1, 1), (0, 0)))

    ocat = jnp.concatenate(
        [r1p[ky:ky + TH, kx:kx + W, :] for ky in range(3) for kx in range(3)],
        axis=-1).reshape(TH * W, 9 * C)
    acc2 = jax.lax.dot_general(wo_ref[...], ocat, (((1,), (1,)), ((), ())),
                               preferred_element_type=jnp.float32)
    o_ref[0] = jnp.maximum(acc2 + bo_ref[0], 0.0).reshape(256, TH, W)


def _fold_bn(w, g, b, m, v):
    s = g / jnp.sqrt(v + EPS)
    w2 = (w * s[:, None, None, None]).transpose(2, 3, 1, 0).astype(jnp.bfloat16)
    return w2, (b - m * s)[None, :]


def kernel(x, w_p1, g_p1, b_p1, m_p1, v_p1, w_p2, g_p2, b_p2, m_p2, v_p2,
           w_p, g_pbn, b_pbn, m_pbn, v_pbn, w_c1, g_bn1, b_bn1, m_bn1, v_bn1,
           w_c2, g_c2, b_c2, m_c2, v_c2):
    xp = jnp.pad(x.transpose(0, 2, 3, 1).astype(jnp.bfloat16),
                 ((0, 0), (1, 1), (1, 1), (0, 0)))

    w1, bb1 = _fold_bn(w_p1, g_p1, b_p1, m_p1, v_p1)
    w2, bb2 = _fold_bn(w_p2, g_p2, b_p2, m_p2, v_p2)
    wb = jnp.concatenate([w1, w2], axis=-1).reshape(9 * C, 256)
    bb = jnp.concatenate([bb1, bb2], axis=-1)
    wm, bm = _fold_bn(w_p, g_pbn, b_pbn, m_pbn, v_pbn)
    wm = wm.reshape(9 * 128, 256)
    ws, bs = _fold_bn(w_c1, g_bn1, b_bn1, m_bn1, v_bn1)
    ws = ws.reshape(C, 256)
    wo, bo = _fold_bn(w_c2, g_c2, b_c2, m_c2, v_c2)
    wo = wo.reshape(9 * C, 256).T
    bo = bo.reshape(256, 1)

    cparams = pltpu.CompilerParams(
        dimension_semantics=("parallel", "arbitrary"),
        vmem_limit_bytes=64 * 1024 * 1024,
    )

    full = lambda *shape: pl.BlockSpec(shape, lambda b_, i: (0,) * len(shape))

    cm, rm = pl.pallas_call(
        _k1,
        grid=(B, NT),
        in_specs=[
            pl.BlockSpec((1, H + 2, W + 2, C), lambda b_, i: (b_, 0, 0, 0)),
            full(9 * C, 256),
            full(1, 256),
        ],
        out_specs=[
            pl.BlockSpec((1, W, 128), lambda b_, i: (b_, 0, 0)),
            pl.BlockSpec((1, TH, 128), lambda b_, i: (b_, i, 0)),
        ],
        out_shape=[
            jax.ShapeDtypeStruct((B, W, 128), jnp.float32),
            jax.ShapeDtypeStruct((B, H, 128), jnp.float32),
        ],
        compiler_params=cparams,
    )(xp, wb, bb)

    rmp = jnp.pad(rm, ((0, 0), (2, 2), (0, 0)))

    out_t = pl.pallas_call(
        _k2,
        grid=(B, NT),
        in_specs=[
            pl.BlockSpec((1, H + 2, W + 2, C), lambda b_, i: (b_, 0, 0, 0)),
            pl.BlockSpec((1, W, 128), lambda b_, i: (b_, 0, 0)),
            pl.BlockSpec((1, H + 4, 128), lambda b_, i: (b_, 0, 0)),
            full(9 * 128, 256),
            full(1, 256),
            full(C, 256),
            full(1, 256),
            full(256, 9 * C),
            full(256, 1),
        ],
        out_specs=pl.BlockSpec((1, 256, TH, W), lambda b_, i: (b_, 0, i, 0)),
        out_shape=jax.ShapeDtypeStruct((B, 256, H, W), jnp.float32),
        compiler_params=cparams,
    )(xp, cm, rmp, wm, bm, ws, bs, wo, bo)

    return out_t
